# baseline (speedup 1.0000x reference)
import functools

import jax
import jax.numpy as jnp
from jax import lax
from jax.experimental import pallas as pl
from jax.experimental.pallas import tpu as pltpu
from jax.experimental.pallas import tpu_sc as plsc

EMBED = 128
S = 25
B_PAD = 10240
NC, NS = 2, 16
NW = NC * NS
RW = B_PAD // NW
CH = 5
NCHUNK = RW // CH
SELF_CH = 64
NSELF = RW // SELF_CH
NBUF = 6
WPR = EMBED // 2


def _sc_gather_body(embw, nodes_r, n0_r, n1_r, n2_r,
                    out_self, out_n0, out_n1, out_n2,
                    idx0, idx1, sidx_v, swbuf, sbuf,
                    buf0, buf1, buf2, buf3, buf4, buf5, stage,
                    sem0, sem1, sem2, sem3, sem4, sem5):
    wid = lax.axis_index("s") * NC + lax.axis_index("c")
    base = wid * RW
    bufs = (buf0, buf1, buf2, buf3, buf4, buf5)
    sems = (sem0, sem1, sem2, sem3, sem4, sem5)

    nv = WPR // 16

    def load_row(buf, rowi):
        out = []
        for v in range(nv):
            w = buf[rowi, pl.ds(v * 16, 16)]
            e, o = plsc.unpack(plsc.bitcast(w, jnp.bfloat16),
                               format=plsc.PackFormat.INTERLEAVED)
            out += [e, o]
        return tuple(out)

    def store_row(dst, rowi, vecs):
        for v in range(nv):
            dst[rowi, pl.ds(v * 16, 16)] = vecs[2 * v]
            dst[rowi, pl.ds(WPR + v * 16, 16)] = vecs[2 * v + 1]

    pltpu.sync_copy(nodes_r.at[wid], sidx_v)

    def self_chunk(ch, carry):
        pltpu.sync_copy(embw.at[sidx_v.at[ch]], swbuf)

        def srow(r, rcarry):
            store_row(sbuf, r, load_row(swbuf, r))
            return rcarry

        lax.fori_loop(0, SELF_CH, srow, 0)
        pltpu.sync_copy(sbuf,
                        out_self.at[pl.ds(base + ch * SELF_CH, SELF_CH)])
        return carry

    lax.fori_loop(0, NSELF, self_chunk, 0)

    pltpu.sync_copy(n0_r.at[wid], idx0)
    pltpu.sync_copy(n1_r.at[wid], idx1)

    def accum(buf, c):
        def row(r, rcarry):
            b0 = r * S
            init = load_row(buf, b0)

            def add_n(n, acc):
                nxt = load_row(buf, b0 + n)
                return tuple(a + x for a, x in zip(acc, nxt))

            acc = lax.fori_loop(1, S, add_n, init)
            store_row(stage, c * CH + r, acc)
            return rcarry
        lax.fori_loop(0, CH, row, 0)

    for li, (idx_v, out_n) in enumerate(
            ((idx0, out_n0), (idx1, out_n1), (idx0, out_n2))):
        if li == 2:
            pltpu.sync_copy(n2_r.at[wid], idx0)
        for b in range(NBUF):
            pltpu.async_copy(embw.at[idx_v.at[b]], bufs[b], sems[b])

        def group(g, carry):
            c0 = g * NBUF
            for b in range(NBUF):
                c = c0 + b
                pltpu.make_async_copy(
                    embw.at[idx_v.at[c]], bufs[b], sems[b]).wait()
                accum(bufs[b], c)

                @pl.when(c + NBUF < NCHUNK)
                def _():
                    pltpu.async_copy(
                        embw.at[idx_v.at[c + NBUF]], bufs[b], sems[b])
            return carry

        lax.fori_loop(0, NCHUNK // NBUF, group, 0)
        pltpu.sync_copy(stage, out_n.at[pl.ds(base, RW)])


def _sc_gather(embw, nodes_r, n0_r, n1_r, n2_r):
    mesh = plsc.VectorSubcoreMesh(core_axis_name="c", subcore_axis_name="s")
    f32 = jnp.float32
    out = jax.ShapeDtypeStruct((B_PAD, EMBED), f32)
    call = pl.kernel(
        _sc_gather_body, mesh=mesh,
        compiler_params=pltpu.CompilerParams(
            needs_layout_passes=False, use_tc_tiling_on_sc=False),
        out_type=[out, out, out, out],
        scratch_types=[
            pltpu.VMEM((NCHUNK, CH * S), jnp.int32),
            pltpu.VMEM((NCHUNK, CH * S), jnp.int32),
            pltpu.VMEM((NSELF, SELF_CH), jnp.int32),
            pltpu.VMEM((SELF_CH, WPR), f32),
            pltpu.VMEM((SELF_CH, EMBED), f32),
        ] + [pltpu.VMEM((CH * S, WPR), f32)
             for _ in range(NBUF)]
        + [pltpu.VMEM((RW, EMBED), f32)]
        + [pltpu.SemaphoreType.DMA] * NBUF,
    )
    return call(embw, nodes_r, n0_r, n1_r, n2_r)


def _tc_dense_body(sv_ref, n0_ref, n1_ref, n2_ref,
                   ws0_ref, wn0_ref, ws1_ref, wn1_ref, ws2_ref, wn2_ref,
                   dense_ref, out_ref):
    inv_s = jnp.float32(1.0 / S)
    dot = functools.partial(jnp.dot, preferred_element_type=jnp.float32)
    h = sv_ref[...]
    for nref, ws, wn in ((n0_ref, ws0_ref, wn0_ref),
                         (n1_ref, ws1_ref, wn1_ref),
                         (n2_ref, ws2_ref, wn2_ref)):
        mean = nref[...] * inv_s
        h = jnp.concatenate([dot(h, ws[...]), dot(mean, wn[...])], axis=-1)
        h = jnp.maximum(h, 0.0)
    out_ref[...] = jax.nn.sigmoid(dot(h, dense_ref[...]))


def _tc_dense(sv, n0, n1, n2, ws0, wn0, ws1, wn1, ws2, wn2, dense):
    bt = 512
    grid = (B_PAD // bt,)
    row_spec = pl.BlockSpec((bt, EMBED), lambda i: (i, 0))
    full = lambda a: pl.BlockSpec(a.shape, lambda i: (0,) * a.ndim)
    return pl.pallas_call(
        _tc_dense_body,
        grid=grid,
        in_specs=[row_spec, row_spec, row_spec, row_spec,
                  full(ws0), full(wn0), full(ws1), full(wn1),
                  full(ws2), full(wn2), full(dense)],
        out_specs=pl.BlockSpec((bt, 64), lambda i: (i, 0)),
        out_shape=jax.ShapeDtypeStruct((B_PAD, 64), jnp.float32),
    )(sv, n0, n1, n2, ws0, wn0, ws1, wn1, ws2, wn2, dense)


def kernel(nodes, neigh0, neigh1, neigh2, embedding,
           ws0, wn0, ws1, wn1, ws2, wn2, dense):
    b = nodes.shape[0]
    pad = B_PAD - b
    nodes_p = jnp.pad(nodes.astype(jnp.int32), (0, pad))
    nodes_r = nodes_p.reshape(NW, NSELF, SELF_CH)
    neigh_r = [
        jnp.pad(n.astype(jnp.int32), ((0, pad), (0, 0)))
        .reshape(NW, NCHUNK, CH * S)
        for n in (neigh0, neigh1, neigh2)
    ]
    embw = jax.lax.bitcast_convert_type(
        embedding.astype(jnp.bfloat16).reshape(-1, WPR, 2), jnp.float32)
    perm = jnp.concatenate([
        jnp.arange(32 * v + p, 32 * v + 32 + p, 2)
        for p in (0, 1) for v in range(WPR // 16)])
    sv, s0, s1, s2 = _sc_gather(embw, nodes_r, *neigh_r)
    out = _tc_dense(sv, s0, s1, s2, ws0[perm], wn0[perm], ws1, wn1[perm],
                    ws2, wn2[perm], dense)
    return out[:b]


# speedup vs baseline: 1.4772x; 1.4772x over previous
import functools

import jax
import jax.numpy as jnp
from jax import lax
from jax.experimental import pallas as pl
from jax.experimental.pallas import tpu as pltpu
from jax.experimental.pallas import tpu_sc as plsc

EMBED = 128
S = 25
B_PAD = 10240
NC, NS = 2, 16
NW = NC * NS
RW = B_PAD // NW
CH = 5
KK = 1
NCHUNK = RW // (CH * KK)
SELF_CH = 64
NSELF = RW // SELF_CH
NBUF = 4


def _sc_gather_body(emb, nodes_r, n0_r, n1_r, n2_r,
                    out_self, out_n0, out_n1, out_n2,
                    idx0, idx1, sidx_v,
                    buf0, buf1, buf2, buf3, stage,
                    sem0, sem1, sem2, sem3):
    wid = lax.axis_index("s") * NC + lax.axis_index("c")
    base = wid * RW
    bufs = (buf0, buf1, buf2, buf3)
    sems = (sem0, sem1, sem2, sem3)

    pltpu.sync_copy(nodes_r.at[wid], sidx_v)

    def self_chunk(ch, carry):
        pltpu.sync_copy(emb.at[sidx_v.at[ch]], buf0.at[pl.ds(0, SELF_CH)])
        pltpu.sync_copy(buf0.at[pl.ds(0, SELF_CH)],
                        out_self.at[pl.ds(base + ch * SELF_CH, SELF_CH)])
        return carry

    lax.fori_loop(0, NSELF, self_chunk, 0)

    pltpu.sync_copy(n0_r.at[wid], idx0)
    pltpu.sync_copy(n1_r.at[wid], idx1)

    nv = EMBED // 16

    def accum(buf, c):
        def row(r, rcarry):
            b0 = r * S
            init = tuple(buf[b0, pl.ds(v * 16, 16)] for v in range(nv))

            def add_n(n, acc):
                return tuple(acc[v] + buf[b0 + n, pl.ds(v * 16, 16)]
                             for v in range(nv))

            acc = lax.fori_loop(1, S, add_n, init)
            for v in range(nv):
                stage[c * KK * CH + r, pl.ds(v * 16, 16)] = acc[v]
            return rcarry
        lax.fori_loop(0, KK * CH, row, 0)

    for li, (idx_v, out_n) in enumerate(
            ((idx0, out_n0), (idx1, out_n1), (idx0, out_n2))):
        if li == 2:
            pltpu.sync_copy(n2_r.at[wid], idx0)
        for b in range(NBUF):
            pltpu.async_copy(emb.at[idx_v.at[b]], bufs[b], sems[b])

        def group(g, carry):
            c0 = g * NBUF
            for b in range(NBUF):
                c = c0 + b
                pltpu.make_async_copy(
                    emb.at[idx_v.at[c]], bufs[b], sems[b]).wait()
                accum(bufs[b], c)

                @pl.when(c + NBUF < NCHUNK)
                def _():
                    pltpu.async_copy(
                        emb.at[idx_v.at[c + NBUF]], bufs[b], sems[b])
            return carry

        lax.fori_loop(0, NCHUNK // NBUF, group, 0)
        pltpu.sync_copy(stage, out_n.at[pl.ds(base, RW)])


def _sc_gather(emb, nodes_r, n0_r, n1_r, n2_r):
    mesh = plsc.VectorSubcoreMesh(core_axis_name="c", subcore_axis_name="s")
    f32 = jnp.float32
    out = jax.ShapeDtypeStruct((B_PAD, EMBED), f32)
    call = pl.kernel(
        _sc_gather_body, mesh=mesh,
        out_type=[out, out, out, out],
        scratch_types=[
            pltpu.VMEM((NCHUNK, KK * CH * S), jnp.int32),
            pltpu.VMEM((NCHUNK, KK * CH * S), jnp.int32),
            pltpu.VMEM((NSELF, SELF_CH), jnp.int32),
            pltpu.VMEM((KK * CH * S, EMBED), f32),
            pltpu.VMEM((KK * CH * S, EMBED), f32),
            pltpu.VMEM((KK * CH * S, EMBED), f32),
            pltpu.VMEM((KK * CH * S, EMBED), f32),
            pltpu.VMEM((RW, EMBED), f32),
            pltpu.SemaphoreType.DMA,
            pltpu.SemaphoreType.DMA,
            pltpu.SemaphoreType.DMA,
            pltpu.SemaphoreType.DMA,
        ],
    )
    return call(emb, nodes_r, n0_r, n1_r, n2_r)


def _tc_dense_body(sv_ref, n0_ref, n1_ref, n2_ref,
                   ws0_ref, wn0_ref, ws1_ref, wn1_ref, ws2_ref, wn2_ref,
                   dense_ref, out_ref):
    inv_s = jnp.float32(1.0 / S)
    dot = functools.partial(jnp.dot, preferred_element_type=jnp.float32)
    h = sv_ref[...]
    for nref, ws, wn in ((n0_ref, ws0_ref, wn0_ref),
                         (n1_ref, ws1_ref, wn1_ref),
                         (n2_ref, ws2_ref, wn2_ref)):
        mean = nref[...] * inv_s
        h = jnp.concatenate([dot(h, ws[...]), dot(mean, wn[...])], axis=-1)
        h = jnp.maximum(h, 0.0)
    out_ref[...] = jax.nn.sigmoid(dot(h, dense_ref[...]))


def _tc_dense(sv, n0, n1, n2, ws0, wn0, ws1, wn1, ws2, wn2, dense):
    bt = 512
    grid = (B_PAD // bt,)
    row_spec = pl.BlockSpec((bt, EMBED), lambda i: (i, 0))
    full = lambda a: pl.BlockSpec(a.shape, lambda i: (0,) * a.ndim)
    return pl.pallas_call(
        _tc_dense_body,
        grid=grid,
        in_specs=[row_spec, row_spec, row_spec, row_spec,
                  full(ws0), full(wn0), full(ws1), full(wn1),
                  full(ws2), full(wn2), full(dense)],
        out_specs=pl.BlockSpec((bt, 64), lambda i: (i, 0)),
        out_shape=jax.ShapeDtypeStruct((B_PAD, 64), jnp.float32),
    )(sv, n0, n1, n2, ws0, wn0, ws1, wn1, ws2, wn2, dense)


def kernel(nodes, neigh0, neigh1, neigh2, embedding,
           ws0, wn0, ws1, wn1, ws2, wn2, dense):
    b = nodes.shape[0]
    pad = B_PAD - b
    nodes_p = jnp.pad(nodes.astype(jnp.int32), (0, pad))
    nodes_r = nodes_p.reshape(NW, NSELF, SELF_CH)
    neigh_r = [
        jnp.pad(n.astype(jnp.int32), ((0, pad), (0, 0)))
        .reshape(NW, NCHUNK, KK * CH * S)
        for n in (neigh0, neigh1, neigh2)
    ]
    sv, s0, s1, s2 = _sc_gather(embedding, nodes_r, *neigh_r)
    out = _tc_dense(sv, s0, s1, s2, ws0, wn0, ws1, wn1, ws2, wn2, dense)
    return out[:b]
